# baseline (speedup 1.0000x reference)
import dataclasses

import jax
import jax.numpy as jnp
from jax import lax
from jax.experimental import pallas as pl
from jax.experimental.pallas import tpu as pltpu
from jax.experimental.pallas import tpu_sc as plsc

B = 16
N0 = 100000
N1 = 65536
N2 = 16384
K = 128
NCLS = 10
NW = 32
CHUNK = 3200
NPAD = NW * CHUNK
NV = CHUNK // 16


def _segsum_body(x_hbm, a0_hbm, t1_hbm, t2_hbm, out_hbm,
                 xv, i0_v, t1_v, t2_v, acc, sems):
    c = lax.axis_index("core")
    s = lax.axis_index("subcore")
    wid = s * 2 + c
    base = wid * CHUNK

    t1cp = pltpu.async_copy(t1_hbm, t1_v, sems.at[0])
    t2cp = pltpu.async_copy(t2_hbm, t2_v, sems.at[1])
    pltpu.sync_copy(a0_hbm.at[pl.ds(base, CHUNK)], i0_v)
    pltpu.sync_copy(x_hbm.at[:, pl.ds(base, CHUNK)], xv)
    t1cp.wait()
    t2cp.wait()

    zero = jnp.zeros((16,), jnp.float32)

    @pl.loop(0, K // 16)
    def _z(kk):
        for b in range(B):
            acc[b, pl.ds(kk * 16, 16)] = zero

    @pl.loop(0, NV)
    def _main(v):
        off = v * 16
        i0 = i0_v[pl.ds(off, 16)]
        w1 = plsc.load_gather(t1_v, [jnp.right_shift(i0, 1)])
        i1 = jnp.right_shift(w1, jnp.left_shift(i0 & 1, 4)) & 0xFFFF
        w2 = plsc.load_gather(t2_v, [jnp.right_shift(i1, 2)])
        i2 = jnp.right_shift(w2, jnp.left_shift(i1 & 3, 3)) & 0xFF
        for b in range(B):
            bv = jnp.full((16,), b, jnp.int32)
            xb = xv[b, pl.ds(off, 16)]
            plsc.addupdate_scatter(acc, [bv, i2], xb)

    pltpu.sync_copy(acc, out_hbm.at[wid])


def _epilogue(s2, v0_ref, g0_ref, v1_ref, g1_ref, v2_ref, g2_ref,
              m_ref, fcg_ref, fcb_ref, out_ref):
    def wn(v, g):
        n = jnp.sqrt(jnp.sum(v * v, axis=1, keepdims=True))
        return g * v / (n + 1e-12)

    def mm(a, b):
        return lax.dot_general(a, b, (((1,), (0,)), ((), ())),
                               preferred_element_type=jnp.float32)

    w0 = wn(v0_ref[...], g0_ref[...])
    w1 = wn(v1_ref[...], g1_ref[...])
    w2 = wn(v2_ref[...], g2_ref[...])
    w = mm(w2, mm(w1, w0))
    wrep = jnp.concatenate([w] * NCLS, axis=0)
    r0 = lax.broadcasted_iota(jnp.int32, (NCLS * K, NCLS), 0)
    r1 = lax.broadcasted_iota(jnp.int32, (NCLS * K, NCLS), 1)
    sel = (jnp.right_shift(r0, 7) == r1).astype(jnp.float32)
    m = m_ref[...]
    d = lax.dot_general(s2, m, (((1,), (1,)), ((), ())),
                        preferred_element_type=jnp.float32)
    e = mm(d, sel * wrep)
    rowsq = jnp.sum(m * m, axis=1, keepdims=True)
    nsq = mm(jnp.ones((1, NCLS * K), jnp.float32), sel * rowsq)
    scale = fcg_ref[...] / (jnp.sqrt(nsq) + 1e-12)
    out_ref[...] = scale * e + fcb_ref[...]


def _epi_body(p_ref, v0_ref, g0_ref, v1_ref, g1_ref, v2_ref, g2_ref,
              m_ref, fcg_ref, fcb_ref, out_ref):
    s2 = jnp.sum(p_ref[...], axis=0)
    _epilogue(s2, v0_ref, g0_ref, v1_ref, g1_ref, v2_ref, g2_ref,
              m_ref, fcg_ref, fcb_ref, out_ref)


def kernel(x, assign0, assign1, assign2, V0, g0, V1, g1, V2, g2,
           fc_V, fc_g, fc_b):
    a1p = lax.bitcast_convert_type(
        assign1.astype(jnp.int16).reshape(N1 // 2, 2), jnp.int32)
    a2p = lax.bitcast_convert_type(
        assign2.astype(jnp.int8).reshape(N2 // 4, 4), jnp.int32)
    xp = jnp.pad(x, ((0, 0), (0, NPAD - N0)))
    a0p = jnp.pad(assign0, (0, NPAD - N0))

    vector_mesh = plsc.VectorSubcoreMesh(
        core_axis_name="core", subcore_axis_name="subcore")
    sc_params = pltpu.CompilerParams()
    if "needs_layout_passes" in pltpu.CompilerParams.__dataclass_fields__:
        sc_params = dataclasses.replace(sc_params, needs_layout_passes=False)
    partials = pl.kernel(
        out_type=jax.ShapeDtypeStruct((NW, B, K), jnp.float32),
        mesh=vector_mesh,
        scratch_types=[pltpu.VMEM((B, CHUNK), jnp.float32),
                       pltpu.VMEM((CHUNK,), jnp.int32),
                       pltpu.VMEM((N1 // 2,), jnp.int32),
                       pltpu.VMEM((N2 // 4,), jnp.int32),
                       pltpu.VMEM((B, K), jnp.float32),
                       pltpu.SemaphoreType.DMA((2,))],
        compiler_params=sc_params,
    )(_segsum_body)(xp, a0p, a1p, a2p)

    out = pl.pallas_call(
        _epi_body,
        out_shape=jax.ShapeDtypeStruct((B, NCLS), jnp.float32),
    )(partials, V0, g0.reshape(32, 1), V1, g1.reshape(64, 1), V2,
      g2.reshape(128, 1), fc_V.reshape(NCLS * K, K),
      fc_g.reshape(1, NCLS), fc_b.reshape(1, NCLS))
    return out
